# initial kernel scaffold (unmeasured)
import jax
import jax.numpy as jnp
from jax import lax
from jax.experimental import pallas as pl
from jax.experimental.pallas import tpu as pltpu

B, SQ, SKV, H, D = 8, 8, 1024, 16, 128
SCALE = D ** -0.5


def _partials_kernel(q_ref, k_ref, v_ref, o_ref, l_ref):
    q = q_ref[0, :, 0, :]
    k = k_ref[0, :, 0, :]
    v = v_ref[0, :, 0, :]
    s = lax.dot_general(
        q, k, (((1,), (1,)), ((), ())), preferred_element_type=jnp.float32
    ) * SCALE
    p = jnp.exp(s)
    l_ref[0, :, 0] = jnp.sum(p, axis=1)
    o_ref[0, :, 0, :] = lax.dot_general(
        p, v, (((1,), (0,)), ((), ())), preferred_element_type=jnp.float32
    )


def _compute_partials(Q, K, V):
    return pl.pallas_call(
        _partials_kernel,
        grid=(B, H),
        in_specs=[
            pl.BlockSpec((1, SQ, 1, D), lambda b, h: (b, 0, h, 0)),
            pl.BlockSpec((1, SKV, 1, D), lambda b, h: (b, 0, h, 0)),
            pl.BlockSpec((1, SKV, 1, D), lambda b, h: (b, 0, h, 0)),
        ],
        out_specs=[
            pl.BlockSpec((1, SQ, 1, D), lambda b, h: (b, 0, h, 0)),
            pl.BlockSpec((1, SQ, 1), lambda b, h: (b, 0, h)),
        ],
        out_shape=[
            jax.ShapeDtypeStruct((B, SQ, H, D), jnp.float32),
            jax.ShapeDtypeStruct((B, SQ, H), jnp.float32),
        ],
    )(Q, K, V)


def _combine_kernel(
    o_ref, l_ref, out_ref, ro_ref, rl_ref, send_sems, recv_sems
):
    my_x = lax.axis_index("x")
    my_y = lax.axis_index("y")
    peer = (1 - my_x, my_y)

    barrier = pltpu.get_barrier_semaphore()
    pl.semaphore_signal(
        barrier, inc=1, device_id=peer, device_id_type=pl.DeviceIdType.MESH
    )
    pl.semaphore_wait(barrier, 1)

    rdma_o = pltpu.make_async_remote_copy(
        src_ref=o_ref,
        dst_ref=ro_ref,
        send_sem=send_sems.at[0],
        recv_sem=recv_sems.at[0],
        device_id=peer,
        device_id_type=pl.DeviceIdType.MESH,
    )
    rdma_l = pltpu.make_async_remote_copy(
        src_ref=l_ref,
        dst_ref=rl_ref,
        send_sem=send_sems.at[1],
        recv_sem=recv_sems.at[1],
        device_id=peer,
        device_id_type=pl.DeviceIdType.MESH,
    )
    rdma_o.start()
    rdma_l.start()
    rdma_o.wait()
    rdma_l.wait()

    denom = l_ref[...] + rl_ref[...]
    out_ref[...] = (o_ref[...] + ro_ref[...]) / denom[:, :, :, None]


def _exchange_combine(o_num, l):
    return pl.pallas_call(
        _combine_kernel,
        in_specs=[
            pl.BlockSpec(memory_space=pltpu.VMEM),
            pl.BlockSpec(memory_space=pltpu.VMEM),
        ],
        out_specs=pl.BlockSpec(memory_space=pltpu.VMEM),
        out_shape=jax.ShapeDtypeStruct((B, SQ, H, D), jnp.float32),
        scratch_shapes=[
            pltpu.VMEM((B, SQ, H, D), jnp.float32),
            pltpu.VMEM((B, SQ, H), jnp.float32),
            pltpu.SemaphoreType.DMA((2,)),
            pltpu.SemaphoreType.DMA((2,)),
        ],
        compiler_params=pltpu.CompilerParams(collective_id=0),
    )(o_num, l)


def kernel(Q, K, V):
    o_num, l = _compute_partials(Q, K, V)
    return _exchange_combine(o_num, l)


# baseline (device time: 87646 ns/iter reference)
import jax
import jax.numpy as jnp
from jax import lax
from jax.experimental import pallas as pl
from jax.experimental.pallas import tpu as pltpu

B, SQ, SKV, H, D = 8, 8, 1024, 16, 128
HC = 8
SCALE = D ** -0.5


def _partials_kernel(q_ref, k_ref, v_ref, o_ref, l_ref):
    for h in range(HC):
        q = q_ref[0, :, h, :]
        k = k_ref[0, :, h, :]
        v = v_ref[0, :, h, :]
        s = lax.dot_general(
            q, k, (((1,), (1,)), ((), ())), preferred_element_type=jnp.float32
        ) * SCALE
        p = jnp.exp(s)
        l_ref[0, h, :] = jnp.sum(p, axis=1)
        o_ref[0, :, h, :] = lax.dot_general(
            p, v, (((1,), (0,)), ((), ())), preferred_element_type=jnp.float32
        )


def _compute_partials(Q, K, V):
    return pl.pallas_call(
        _partials_kernel,
        grid=(B, H // HC),
        in_specs=[
            pl.BlockSpec((1, SQ, HC, D), lambda b, hc: (b, 0, hc, 0)),
            pl.BlockSpec((1, SKV, HC, D), lambda b, hc: (b, 0, hc, 0)),
            pl.BlockSpec((1, SKV, HC, D), lambda b, hc: (b, 0, hc, 0)),
        ],
        out_specs=[
            pl.BlockSpec((1, SQ, HC, D), lambda b, hc: (b, 0, hc, 0)),
            pl.BlockSpec((1, HC, SQ), lambda b, hc: (b, hc, 0)),
        ],
        out_shape=[
            jax.ShapeDtypeStruct((B, SQ, H, D), jnp.float32),
            jax.ShapeDtypeStruct((B, H, SQ), jnp.float32),
        ],
    )(Q, K, V)


def _combine_kernel(
    o_ref, l_ref, out_ref, ro_ref, rl_ref, send_sems, recv_sems
):
    my_x = lax.axis_index("x")
    my_y = lax.axis_index("y")
    peer = (1 - my_x, my_y)

    barrier = pltpu.get_barrier_semaphore()
    pl.semaphore_signal(
        barrier, inc=1, device_id=peer, device_id_type=pl.DeviceIdType.MESH
    )
    pl.semaphore_wait(barrier, 1)

    rdma_o = pltpu.make_async_remote_copy(
        src_ref=o_ref,
        dst_ref=ro_ref,
        send_sem=send_sems.at[0],
        recv_sem=recv_sems.at[0],
        device_id=peer,
        device_id_type=pl.DeviceIdType.MESH,
    )
    rdma_l = pltpu.make_async_remote_copy(
        src_ref=l_ref,
        dst_ref=rl_ref,
        send_sem=send_sems.at[1],
        recv_sem=recv_sems.at[1],
        device_id=peer,
        device_id_type=pl.DeviceIdType.MESH,
    )
    rdma_o.start()
    rdma_l.start()
    rdma_o.wait()
    rdma_l.wait()

    denom = l_ref[...] + rl_ref[...]
    denom_t = jnp.transpose(denom, (0, 2, 1))
    out_ref[...] = (o_ref[...] + ro_ref[...]) / denom_t[:, :, :, None]


def _exchange_combine(o_num, l):
    return pl.pallas_call(
        _combine_kernel,
        in_specs=[
            pl.BlockSpec(memory_space=pltpu.VMEM),
            pl.BlockSpec(memory_space=pltpu.VMEM),
        ],
        out_specs=pl.BlockSpec(memory_space=pltpu.VMEM),
        out_shape=jax.ShapeDtypeStruct((B, SQ, H, D), jnp.float32),
        scratch_shapes=[
            pltpu.VMEM((B, SQ, H, D), jnp.float32),
            pltpu.VMEM((B, H, SQ), jnp.float32),
            pltpu.SemaphoreType.DMA((2,)),
            pltpu.SemaphoreType.DMA((2,)),
        ],
        compiler_params=pltpu.CompilerParams(collective_id=0),
    )(o_num, l)


def kernel(Q, K, V):
    o_num, l = _compute_partials(Q, K, V)
    return _exchange_combine(o_num, l)
